# baseline (speedup 1.0000x reference)
import jax
import jax.numpy as jnp
from jax.experimental import pallas as pl
from jax.experimental.pallas import tpu as pltpu

_CODEBOOK_SIZE = 1024
_CODE_SIZE = 256
_BETA = 0.25
_M_BLOCK = 2048
_N_STEPS = 2
_CB_SCALE = 512.0


def _half(z, cb8, hcs, ones_k):
    zc = jax.lax.dot_general(
        cb8, z.astype(jnp.float8_e4m3fn),
        dimension_numbers=(((1,), (1,)), ((), ())),
        preferred_element_type=jnp.float32,
    ).astype(jnp.bfloat16)
    zsqr = jax.lax.dot_general(
        ones_k, z * z,
        dimension_numbers=(((1,), (1,)), ((), ())),
        preferred_element_type=jnp.float32,
    )
    mx = jnp.max(zc - hcs, axis=0)
    return ((1.0 + _BETA) * zsqr[0]
            - ((2.0 + 2.0 * _BETA) / _CB_SCALE) * mx.astype(jnp.float32))


def _vq_loss_kernel(za_ref, zb_ref, cb_ref, out_ref, cb8_ref, hcs_ref):
    ones_k = jnp.ones((1, _CODE_SIZE), dtype=jnp.float32)

    @pl.when(pl.program_id(0) == 0)
    def _prep_codebook():
        cb = cb_ref[...]
        cb8_ref[...] = (cb * _CB_SCALE).astype(jnp.float8_e4m3fn)
        hcs_ref[...] = jax.lax.dot_general(
            cb * ((0.5 * _CB_SCALE) * cb), ones_k,
            dimension_numbers=(((1,), (1,)), ((), ())),
            preferred_element_type=jnp.float32,
        ).astype(jnp.bfloat16)

    i = pl.program_id(0)
    cb8 = cb8_ref[...]
    hcs = hcs_ref[...]
    out_ref[pl.ds(i * _M_BLOCK, _M_BLOCK)] = _half(
        za_ref[...], cb8, hcs, ones_k)
    out_ref[pl.ds((_N_STEPS + i) * _M_BLOCK, _M_BLOCK)] = _half(
        zb_ref[...], cb8, hcs, ones_k)


@jax.jit
def kernel(z_e_x, codebook):
    batch = z_e_x.shape[0]
    return pl.pallas_call(
        _vq_loss_kernel,
        grid=(_N_STEPS,),
        in_specs=[
            pl.BlockSpec((_M_BLOCK, _CODE_SIZE), lambda i: (i, 0)),
            pl.BlockSpec((_M_BLOCK, _CODE_SIZE), lambda i: (i + _N_STEPS, 0)),
            pl.BlockSpec((_CODEBOOK_SIZE, _CODE_SIZE), lambda i: (0, 0)),
        ],
        out_specs=pl.BlockSpec((batch,), lambda i: (0,)),
        out_shape=jax.ShapeDtypeStruct((batch,), jnp.float32),
        scratch_shapes=[
            pltpu.VMEM((_CODEBOOK_SIZE, _CODE_SIZE), jnp.float8_e4m3fn),
            pltpu.VMEM((_CODEBOOK_SIZE, 1), jnp.bfloat16),
        ],
    )(z_e_x, z_e_x, codebook)


# speedup vs baseline: 1.0210x; 1.0210x over previous
import jax
import jax.numpy as jnp
from jax.experimental import pallas as pl
from jax.experimental.pallas import tpu as pltpu

_CODEBOOK_SIZE = 1024
_CODE_SIZE = 256
_BETA = 0.25
_M_BLOCK = 4096
_CB_SCALE = 512.0


def _vq_loss_kernel(z_ref, cb_ref, out_ref, cb8_ref, hcs_ref):
    ones_k = jnp.ones((1, _CODE_SIZE), dtype=jnp.float32)

    @pl.when(pl.program_id(0) == 0)
    def _prep_codebook():
        cb = cb_ref[...]
        cb8_ref[...] = (cb * _CB_SCALE).astype(jnp.float8_e4m3fn)
        hcs_ref[...] = jax.lax.dot_general(
            cb * ((0.5 * _CB_SCALE) * cb), ones_k,
            dimension_numbers=(((1,), (1,)), ((), ())),
            preferred_element_type=jnp.float32,
        ).astype(jnp.bfloat16)

    z = z_ref[...]
    zc = jax.lax.dot_general(
        cb8_ref[...], z.astype(jnp.float8_e4m3fn),
        dimension_numbers=(((1,), (1,)), ((), ())),
        preferred_element_type=jnp.float32,
    ).astype(jnp.bfloat16)
    zsqr = jax.lax.dot_general(
        ones_k, z * z,
        dimension_numbers=(((1,), (1,)), ((), ())),
        preferred_element_type=jnp.float32,
    )
    mx = jnp.max(zc - hcs_ref[...], axis=0)
    loss = ((1.0 + _BETA) * zsqr[0]
            - ((2.0 + 2.0 * _BETA) / _CB_SCALE) * mx.astype(jnp.float32))
    out_ref[pl.ds(pl.program_id(0) * _M_BLOCK, _M_BLOCK)] = loss


@jax.jit
def kernel(z_e_x, codebook):
    batch = z_e_x.shape[0]
    grid = (batch // _M_BLOCK,)
    return pl.pallas_call(
        _vq_loss_kernel,
        grid=grid,
        in_specs=[
            pl.BlockSpec((_M_BLOCK, _CODE_SIZE), lambda i: (i, 0)),
            pl.BlockSpec((_CODEBOOK_SIZE, _CODE_SIZE), lambda i: (0, 0)),
        ],
        out_specs=pl.BlockSpec((batch,), lambda i: (0,)),
        out_shape=jax.ShapeDtypeStruct((batch,), jnp.float32),
        scratch_shapes=[
            pltpu.VMEM((_CODEBOOK_SIZE, _CODE_SIZE), jnp.float8_e4m3fn),
            pltpu.VMEM((_CODEBOOK_SIZE, 1), jnp.bfloat16),
        ],
    )(z_e_x, codebook)


# speedup vs baseline: 1.0327x; 1.0114x over previous
import jax
import jax.numpy as jnp
from jax.experimental import pallas as pl

_CODEBOOK_SIZE = 1024
_CODE_SIZE = 256
_BETA = 0.25
_M_BLOCK = 4096
_CB_SCALE = 512.0


def _vq_loss_kernel(z_ref, cb_ref, out_ref):
    ones_k = jnp.ones((1, _CODE_SIZE), dtype=jnp.float32)
    cb = cb_ref[...]
    z = z_ref[...]
    zc = jax.lax.dot_general(
        (cb * _CB_SCALE).astype(jnp.float8_e4m3fn),
        z.astype(jnp.float8_e4m3fn),
        dimension_numbers=(((1,), (1,)), ((), ())),
        preferred_element_type=jnp.float32,
    ).astype(jnp.bfloat16)
    hcs = jax.lax.dot_general(
        cb * ((0.5 * _CB_SCALE) * cb), ones_k,
        dimension_numbers=(((1,), (1,)), ((), ())),
        preferred_element_type=jnp.float32,
    ).astype(jnp.bfloat16)
    zsqr = jax.lax.dot_general(
        ones_k, z * z,
        dimension_numbers=(((1,), (1,)), ((), ())),
        preferred_element_type=jnp.float32,
    )
    mx = jnp.max(zc - hcs, axis=0)
    loss = ((1.0 + _BETA) * zsqr[0]
            - ((2.0 + 2.0 * _BETA) / _CB_SCALE) * mx.astype(jnp.float32))
    out_ref[pl.ds(pl.program_id(0) * _M_BLOCK, _M_BLOCK)] = loss


@jax.jit
def kernel(z_e_x, codebook):
    batch = z_e_x.shape[0]
    grid = (batch // _M_BLOCK,)
    return pl.pallas_call(
        _vq_loss_kernel,
        grid=grid,
        in_specs=[
            pl.BlockSpec((_M_BLOCK, _CODE_SIZE), lambda i: (i, 0)),
            pl.BlockSpec((_CODEBOOK_SIZE, _CODE_SIZE), lambda i: (0, 0)),
        ],
        out_specs=pl.BlockSpec((batch,), lambda i: (0,)),
        out_shape=jax.ShapeDtypeStruct((batch,), jnp.float32),
    )(z_e_x, codebook)
